# Initial kernel scaffold; baseline (speedup 1.0000x reference)
import jax
import jax.numpy as jnp
from jax.experimental import pallas as pl


def kernel(x, edge_index, W1, b1, W2, b2):
    raise NotImplementedError("write your pallas kernel here")


# speedup vs baseline: 9.6606x; 9.6606x over previous
import functools

import jax
import jax.numpy as jnp
from jax import lax
from jax.experimental import pallas as pl
from jax.experimental.pallas import tpu as pltpu
from jax.experimental.pallas import tpu_sc as plsc

_NC = 2
_NS = 16
_L = 16
_CHUNK = 80
_RPT = 640
_NP = _NS * _RPT


def _mesh():
    return plsc.VectorSubcoreMesh(core_axis_name="c", subcore_axis_name="s")


@functools.cache
def _deg_kernel(ept):
    cpt = ept // _CHUNK

    @functools.partial(
        pl.kernel,
        out_type=jax.ShapeDtypeStruct((_NC, _NP), jnp.float32),
        mesh=_mesh(),
        scratch_types=[
            pltpu.VMEM((ept,), jnp.int32),
            pltpu.VMEM((_CHUNK,), jnp.int32),
            pltpu.VMEM((_CHUNK,), jnp.float32),
            pltpu.VMEM((_RPT,), jnp.float32),
            pltpu.VMEM_SHARED((_NP,), jnp.float32),
        ],
    )
    def deg(dst_e, out, idx_v, chunk_v, ones_v, zeros_v, acc):
        c = lax.axis_index("c")
        s = lax.axis_index("s")
        tid = c * _NS + s
        for i in range(_RPT // _L):
            zeros_v[pl.ds(i * _L, _L)] = jnp.zeros((_L,), jnp.float32)
        for i in range(_CHUNK // _L):
            ones_v[pl.ds(i * _L, _L)] = jnp.ones((_L,), jnp.float32)
        pltpu.sync_copy(zeros_v, acc.at[pl.ds(s * _RPT, _RPT)])
        pltpu.sync_copy(dst_e.at[pl.ds(tid * ept, ept)], idx_v)
        plsc.subcore_barrier()

        def body(j, carry):
            for k in range(_CHUNK // _L):
                sl = pl.ds(k * _L, _L)
                chunk_v[sl] = idx_v[pl.ds(j * _CHUNK + k * _L, _L)]
            pltpu.sync_copy(ones_v, acc.at[chunk_v], add=True)
            return carry

        lax.fori_loop(0, cpt, body, 0)
        plsc.subcore_barrier()
        pltpu.sync_copy(acc.at[pl.ds(s * _RPT, _RPT)],
                        out.at[c, pl.ds(s * _RPT, _RPT)])

    return deg


_SUP = 32


@functools.cache
def _agg_kernel(n, ept, f, split_edges):
    tpt = ept if split_edges else 2 * ept
    sup_e = _SUP * _CHUNK
    nsup = tpt // sup_e
    last = n - (_NS - 1) * _RPT

    @functools.partial(
        pl.kernel,
        out_type=jax.ShapeDtypeStruct((_NC, _NP, f), jnp.float32),
        mesh=_mesh(),
        scratch_types=[
            pltpu.VMEM((sup_e,), jnp.int32),
            pltpu.VMEM((sup_e,), jnp.int32),
            pltpu.VMEM((_CHUNK,), jnp.int32),
            pltpu.VMEM((_CHUNK,), jnp.int32),
            pltpu.VMEM((_CHUNK, f), jnp.float32),
            pltpu.VMEM_SHARED((_NP, f), jnp.float32),
            pltpu.SemaphoreType.DMA,
        ],
    )
    def agg(xs, src_e, dst_e, out, src_v, dst_v, adj_v, chunk_v, rows_v,
            acc, sem):
        c = lax.axis_index("c")
        s = lax.axis_index("s")
        base = 0 if split_edges else c * n
        ebase = (c * _NS + s) * ept if split_edges else s * tpt

        @pl.when(s < _NS - 1)
        def _():
            pltpu.sync_copy(xs.at[pl.ds(base + s * _RPT, _RPT)],
                            acc.at[pl.ds(s * _RPT, _RPT)])

        @pl.when(s == _NS - 1)
        def _():
            pltpu.sync_copy(xs.at[pl.ds(base + (_NS - 1) * _RPT, last)],
                            acc.at[pl.ds((_NS - 1) * _RPT, last)])

        plsc.subcore_barrier()

        def outer(o, carry):
            off = ebase + o * sup_e
            pltpu.sync_copy(src_e.at[pl.ds(off, sup_e)], src_v)
            pltpu.sync_copy(dst_e.at[pl.ds(off, sup_e)], dst_v)

            def body(j, carry2):
                for k in range(_CHUNK // _L):
                    sl = pl.ds(k * _L, _L)
                    esl = pl.ds(j * _CHUNK + k * _L, _L)
                    adj_v[sl] = src_v[esl] + base
                    chunk_v[sl] = dst_v[esl]
                pltpu.async_copy(xs.at[adj_v], rows_v, sem).wait()
                pltpu.sync_copy(rows_v, acc.at[chunk_v], add=True)
                return carry2

            return lax.fori_loop(0, _SUP, body, carry)

        lax.fori_loop(0, nsup, outer, 0)
        plsc.subcore_barrier()
        pltpu.sync_copy(acc.at[pl.ds(s * _RPT, _RPT)],
                        out.at[c, pl.ds(s * _RPT, _RPT)])

    return agg


def _prep(x, deg_t):
    n, fin = x.shape
    r = 1000
    nb = n // r

    def body(x_ref, d_ref, xs_ref, dinv_ref):
        deg = d_ref[:, 0:1] + d_ref[:, 1:2] + 1.0
        dinv = lax.rsqrt(jnp.maximum(deg, 1.0))
        xs_ref[...] = x_ref[...] * dinv
        dinv_ref[...] = dinv

    return pl.pallas_call(
        body,
        grid=(nb,),
        in_specs=[
            pl.BlockSpec((r, fin), lambda rb: (rb, 0)),
            pl.BlockSpec((r, 2), lambda rb: (rb, 0)),
        ],
        out_specs=[
            pl.BlockSpec((r, fin), lambda rb: (rb, 0)),
            pl.BlockSpec((r, 1), lambda rb: (rb, 0)),
        ],
        out_shape=[
            jax.ShapeDtypeStruct((n, fin), jnp.float32),
            jax.ShapeDtypeStruct((n, 1), jnp.float32),
        ],
    )(x, deg_t)


def _mid(s1p, xs, dinv, w, b):
    n, fin = xs.shape
    r = 1000
    nb = n // r
    fout = w.shape[1]
    foh = fout // 2

    def body(s_ref, x_ref, d_ref, w_ref, b_ref, o_ref):
        a = (s_ref[0] + s_ref[1] - x_ref[...]) * d_ref[...]
        h = jnp.dot(a, w_ref[...], preferred_element_type=jnp.float32)
        h = jnp.maximum(h + b_ref[...], 0.0) * d_ref[...]
        o_ref[0] = h[:, :foh]
        o_ref[1] = h[:, foh:]

    return pl.pallas_call(
        body,
        grid=(nb,),
        in_specs=[
            pl.BlockSpec((2, r, fin), lambda rb: (0, rb, 0)),
            pl.BlockSpec((r, fin), lambda rb: (rb, 0)),
            pl.BlockSpec((r, 1), lambda rb: (rb, 0)),
            pl.BlockSpec(w.shape, lambda rb: (0, 0)),
            pl.BlockSpec((1, fout), lambda rb: (0, 0)),
        ],
        out_specs=pl.BlockSpec((2, r, foh), lambda rb: (0, rb, 0)),
        out_shape=jax.ShapeDtypeStruct((2, n, foh), jnp.float32),
    )(s1p, xs, dinv, w, b)


def _out(s2, dinv, w, b):
    n = dinv.shape[0]
    fh = s2.shape[2]
    r = 1000
    nb = n // r
    fout = w.shape[1]

    def body(s_ref, d_ref, w_ref, b_ref, o_ref):
        a = jnp.concatenate([s_ref[0], s_ref[1]], axis=1) * d_ref[...]
        h = jnp.dot(a, w_ref[...], preferred_element_type=jnp.float32)
        o_ref[...] = jnp.maximum(h + b_ref[...], 0.0)

    return pl.pallas_call(
        body,
        grid=(nb,),
        in_specs=[
            pl.BlockSpec((2, r, fh), lambda rb: (0, rb, 0)),
            pl.BlockSpec((r, 1), lambda rb: (rb, 0)),
            pl.BlockSpec(w.shape, lambda rb: (0, 0)),
            pl.BlockSpec((1, fout), lambda rb: (0, 0)),
        ],
        out_specs=pl.BlockSpec((r, fout), lambda rb: (rb, 0)),
        out_shape=jax.ShapeDtypeStruct((n, fout), jnp.float32),
    )(s2, dinv, w, b)


def kernel(x, edge_index, W1, b1, W2, b2):
    n, fin = x.shape
    e = edge_index.shape[1]
    ei = edge_index.astype(jnp.int32)

    tiles = _NC * _NS
    base_ept = e // tiles
    ept = -(-base_ept // (_SUP * _CHUNK)) * (_SUP * _CHUNK)
    pad = ept - base_ept
    assert pad <= _NP - n, "edge padding must fit accumulator pad rows"
    ei3 = ei.reshape(2, tiles, base_ept)
    src_pad = jnp.concatenate(
        [ei3[0], jnp.zeros((tiles, pad), jnp.int32)], axis=1).reshape(-1)
    dst_fill = n + jnp.arange(pad, dtype=jnp.int32)
    dst_pad = jnp.concatenate(
        [ei3[1], jnp.broadcast_to(dst_fill, (tiles, pad))],
        axis=1).reshape(-1)

    deg_part = _deg_kernel(ept)(dst_pad)
    deg_t = deg_part[:, :n].T
    xs, dinv = _prep(x, deg_t)
    s1p = _agg_kernel(n, ept, fin, True)(xs, src_pad, dst_pad)
    hs = _mid(s1p, xs, dinv, W1, b1.reshape(1, -1))
    fh = W1.shape[1] // 2
    s2 = _agg_kernel(n, ept, fh, False)(hs.reshape(2 * n, fh),
                                        src_pad, dst_pad)
    return _out(s2, dinv, W2, b2.reshape(1, -1))


# speedup vs baseline: 10.9276x; 1.1311x over previous
import functools

import jax
import jax.numpy as jnp
from jax import lax
from jax.experimental import pallas as pl
from jax.experimental.pallas import tpu as pltpu
from jax.experimental.pallas import tpu_sc as plsc

_NC = 2
_NS = 16
_L = 16
_CHUNK = 80
_RPT = 640
_NP = _NS * _RPT


def _mesh():
    return plsc.VectorSubcoreMesh(core_axis_name="c", subcore_axis_name="s")


@functools.cache
def _deg_kernel(ept):
    cpt = ept // _CHUNK

    @functools.partial(
        pl.kernel,
        out_type=jax.ShapeDtypeStruct((_NC, _NP), jnp.float32),
        mesh=_mesh(),
        scratch_types=[
            pltpu.VMEM((ept,), jnp.int32),
            pltpu.VMEM((_CHUNK,), jnp.int32),
            pltpu.VMEM((_CHUNK,), jnp.float32),
            pltpu.VMEM((_RPT,), jnp.float32),
            pltpu.VMEM_SHARED((_NP,), jnp.float32),
        ],
    )
    def deg(dst_e, out, idx_v, chunk_v, ones_v, zeros_v, acc):
        c = lax.axis_index("c")
        s = lax.axis_index("s")
        tid = c * _NS + s
        for i in range(_RPT // _L):
            zeros_v[pl.ds(i * _L, _L)] = jnp.zeros((_L,), jnp.float32)
        for i in range(_CHUNK // _L):
            ones_v[pl.ds(i * _L, _L)] = jnp.ones((_L,), jnp.float32)
        pltpu.sync_copy(zeros_v, acc.at[pl.ds(s * _RPT, _RPT)])
        pltpu.sync_copy(dst_e.at[pl.ds(tid * ept, ept)], idx_v)
        plsc.subcore_barrier()

        def body(j, carry):
            for k in range(_CHUNK // _L):
                sl = pl.ds(k * _L, _L)
                chunk_v[sl] = idx_v[pl.ds(j * _CHUNK + k * _L, _L)]
            pltpu.sync_copy(ones_v, acc.at[chunk_v], add=True)
            return carry

        lax.fori_loop(0, cpt, body, 0)
        plsc.subcore_barrier()
        pltpu.sync_copy(acc.at[pl.ds(s * _RPT, _RPT)],
                        out.at[c, pl.ds(s * _RPT, _RPT)])

    return deg


_SUP = 32


@functools.cache
def _agg_kernel(n, ept, f, split_edges):
    tpt = ept if split_edges else 2 * ept
    sup_e = _SUP * _CHUNK
    nsup = tpt // sup_e
    last = n - (_NS - 1) * _RPT

    total = nsup * _SUP

    @functools.partial(
        pl.kernel,
        out_type=jax.ShapeDtypeStruct((_NC, _NP, f), jnp.float32),
        mesh=_mesh(),
        scratch_types=[
            pltpu.VMEM((sup_e,), jnp.int32),
            pltpu.VMEM((sup_e,), jnp.int32),
            pltpu.VMEM((2, _CHUNK), jnp.int32),
            pltpu.VMEM((2, _CHUNK), jnp.int32),
            pltpu.VMEM((2, _CHUNK, f), jnp.float32),
            pltpu.VMEM_SHARED((_NP, f), jnp.float32),
            pltpu.SemaphoreType.DMA,
            pltpu.SemaphoreType.DMA,
        ],
    )
    def agg(xs, src_e, dst_e, out, src_v, dst_v, adj_v, chunk_v, rows_v,
            acc, gsem, ssem):
        c = lax.axis_index("c")
        s = lax.axis_index("s")
        base = 0 if split_edges else c * n
        ebase = (c * _NS + s) * ept if split_edges else s * tpt

        @pl.when(s < _NS - 1)
        def _():
            pltpu.sync_copy(xs.at[pl.ds(base + s * _RPT, _RPT)],
                            acc.at[pl.ds(s * _RPT, _RPT)])

        @pl.when(s == _NS - 1)
        def _():
            pltpu.sync_copy(xs.at[pl.ds(base + (_NS - 1) * _RPT, last)],
                            acc.at[pl.ds((_NS - 1) * _RPT, last)])

        pltpu.sync_copy(src_e.at[pl.ds(ebase, sup_e)], src_v)
        pltpu.sync_copy(dst_e.at[pl.ds(ebase, sup_e)], dst_v)
        plsc.subcore_barrier()

        def body(g, carry):
            j = g % _SUP
            b = g % 2

            @pl.when((j == 0) & (g > 0))
            def _():
                off = ebase + (g // _SUP) * sup_e
                pltpu.sync_copy(src_e.at[pl.ds(off, sup_e)], src_v)
                pltpu.sync_copy(dst_e.at[pl.ds(off, sup_e)], dst_v)

            @pl.when(g >= 2)
            def _():
                pltpu.make_async_copy(rows_v.at[b], acc.at[chunk_v.at[b]],
                                      ssem).wait()

            for k in range(_CHUNK // _L):
                sl = pl.ds(k * _L, _L)
                esl = pl.ds(j * _CHUNK + k * _L, _L)
                adj_v[b, sl] = src_v[esl] + base
                chunk_v[b, sl] = dst_v[esl]
            pltpu.async_copy(xs.at[adj_v.at[b]], rows_v.at[b], gsem).wait()
            pltpu.async_copy(rows_v.at[b], acc.at[chunk_v.at[b]], ssem,
                             add=True)
            return carry

        lax.fori_loop(0, total, body, 0)
        for b in range(2):
            pltpu.make_async_copy(rows_v.at[b], acc.at[chunk_v.at[b]],
                                  ssem).wait()
        plsc.subcore_barrier()
        pltpu.sync_copy(acc.at[pl.ds(s * _RPT, _RPT)],
                        out.at[c, pl.ds(s * _RPT, _RPT)])

    return agg


def _prep(x, deg_t):
    n, fin = x.shape
    r = 1000
    nb = n // r

    def body(x_ref, d_ref, xs_ref, dinv_ref):
        deg = d_ref[:, 0:1] + d_ref[:, 1:2] + 1.0
        dinv = lax.rsqrt(jnp.maximum(deg, 1.0))
        xs_ref[...] = x_ref[...] * dinv
        dinv_ref[...] = dinv

    return pl.pallas_call(
        body,
        grid=(nb,),
        in_specs=[
            pl.BlockSpec((r, fin), lambda rb: (rb, 0)),
            pl.BlockSpec((r, 2), lambda rb: (rb, 0)),
        ],
        out_specs=[
            pl.BlockSpec((r, fin), lambda rb: (rb, 0)),
            pl.BlockSpec((r, 1), lambda rb: (rb, 0)),
        ],
        out_shape=[
            jax.ShapeDtypeStruct((n, fin), jnp.float32),
            jax.ShapeDtypeStruct((n, 1), jnp.float32),
        ],
    )(x, deg_t)


def _mid(s1p, xs, dinv, w, b):
    n, fin = xs.shape
    r = 1000
    nb = n // r
    fout = w.shape[1]
    foh = fout // 2

    def body(s_ref, x_ref, d_ref, w_ref, b_ref, o_ref):
        a = (s_ref[0] + s_ref[1] - x_ref[...]) * d_ref[...]
        h = jnp.dot(a, w_ref[...], preferred_element_type=jnp.float32)
        h = jnp.maximum(h + b_ref[...], 0.0) * d_ref[...]
        o_ref[0] = h[:, :foh]
        o_ref[1] = h[:, foh:]

    return pl.pallas_call(
        body,
        grid=(nb,),
        in_specs=[
            pl.BlockSpec((2, r, fin), lambda rb: (0, rb, 0)),
            pl.BlockSpec((r, fin), lambda rb: (rb, 0)),
            pl.BlockSpec((r, 1), lambda rb: (rb, 0)),
            pl.BlockSpec(w.shape, lambda rb: (0, 0)),
            pl.BlockSpec((1, fout), lambda rb: (0, 0)),
        ],
        out_specs=pl.BlockSpec((2, r, foh), lambda rb: (0, rb, 0)),
        out_shape=jax.ShapeDtypeStruct((2, n, foh), jnp.float32),
    )(s1p, xs, dinv, w, b)


def _out(s2, dinv, w, b):
    n = dinv.shape[0]
    fh = s2.shape[2]
    r = 1000
    nb = n // r
    fout = w.shape[1]

    def body(s_ref, d_ref, w_ref, b_ref, o_ref):
        a = jnp.concatenate([s_ref[0], s_ref[1]], axis=1) * d_ref[...]
        h = jnp.dot(a, w_ref[...], preferred_element_type=jnp.float32)
        o_ref[...] = jnp.maximum(h + b_ref[...], 0.0)

    return pl.pallas_call(
        body,
        grid=(nb,),
        in_specs=[
            pl.BlockSpec((2, r, fh), lambda rb: (0, rb, 0)),
            pl.BlockSpec((r, 1), lambda rb: (rb, 0)),
            pl.BlockSpec(w.shape, lambda rb: (0, 0)),
            pl.BlockSpec((1, fout), lambda rb: (0, 0)),
        ],
        out_specs=pl.BlockSpec((r, fout), lambda rb: (rb, 0)),
        out_shape=jax.ShapeDtypeStruct((n, fout), jnp.float32),
    )(s2, dinv, w, b)


def kernel(x, edge_index, W1, b1, W2, b2):
    n, fin = x.shape
    e = edge_index.shape[1]
    ei = edge_index.astype(jnp.int32)

    tiles = _NC * _NS
    base_ept = e // tiles
    ept = -(-base_ept // (_SUP * _CHUNK)) * (_SUP * _CHUNK)
    pad = ept - base_ept
    assert pad <= _NP - n, "edge padding must fit accumulator pad rows"
    ei3 = ei.reshape(2, tiles, base_ept)
    src_pad = jnp.concatenate(
        [ei3[0], jnp.zeros((tiles, pad), jnp.int32)], axis=1).reshape(-1)
    dst_fill = n + jnp.arange(pad, dtype=jnp.int32)
    dst_pad = jnp.concatenate(
        [ei3[1], jnp.broadcast_to(dst_fill, (tiles, pad))],
        axis=1).reshape(-1)

    deg_part = _deg_kernel(ept)(dst_pad)
    deg_t = deg_part[:, :n].T
    xs, dinv = _prep(x, deg_t)
    s1p = _agg_kernel(n, ept, fin, True)(xs, src_pad, dst_pad)
    hs = _mid(s1p, xs, dinv, W1, b1.reshape(1, -1))
    fh = W1.shape[1] // 2
    s2 = _agg_kernel(n, ept, fh, False)(hs.reshape(2 * n, fh),
                                        src_pad, dst_pad)
    return _out(s2, dinv, W2, b2.reshape(1, -1))
